# Initial kernel scaffold; baseline (speedup 1.0000x reference)
import jax
import jax.numpy as jnp
from jax.experimental import pallas as pl


def kernel(x_nchw, weight_oihw, gamma, beta):
    raise NotImplementedError("write your pallas kernel here")


# speedup vs baseline: 6.7787x; 6.7787x over previous
import functools
import math

import jax
import jax.numpy as jnp
from jax import lax
from jax.experimental import pallas as pl
from jax.experimental.pallas import tpu as pltpu

_VMEM_LIMIT_BYTES = 64 * 1024 * 1024


def _conv_stats_kernel(x_ref, w_ref, conv_ref, stats_ref, *, nq, qr, w_img,
                       pad_rows):
    r = lax.broadcasted_iota(jnp.int32, (qr, 1), 0) % w_img
    mask_l = r != 0
    mask_r = r != (w_img - 1)
    pieces = []
    for q in range(nq):
        base = pad_rows + q * qr
        for dh in (-1, 0, 1):
            for dw in (-1, 0, 1):
                sl = x_ref[0, pl.ds(base + dh * w_img + dw, qr), :]
                if dw == -1:
                    sl = jnp.where(mask_l, sl, 0.0)
                elif dw == 1:
                    sl = jnp.where(mask_r, sl, 0.0)
                pieces.append(sl)
    patches = jnp.concatenate(pieces, axis=1)
    acc = jnp.dot(patches, w_ref[...], preferred_element_type=jnp.float32)
    conv_ref[0] = acc
    s1 = jnp.sum(acc, axis=0, keepdims=True)
    s2 = jnp.sum(acc * acc, axis=0, keepdims=True)
    stats_ref[0] = jnp.concatenate([s1, s2], axis=0)


def _bn_relu_kernel(conv_ref, scale_ref, shift_ref, o_ref):
    o_ref[...] = jnp.maximum(conv_ref[...] * scale_ref[...] + shift_ref[...], 0.0)


@jax.jit
def _conv_block(x_nchw, weight_oihw, gamma, beta):
    N, Cin, H, W = x_nchw.shape
    Cout, _, KH, KW = weight_oihw.shape
    rows = H * W
    Q = 4 if rows % 4 == 0 else 1
    qr = rows // Q
    pad_rows = W + 1
    P = Q * Cout

    x_t = jnp.transpose(x_nchw, (0, 2, 3, 1)).reshape(N, rows, Cin)
    x_p = jnp.pad(x_t, ((0, 0), (pad_rows, pad_rows), (0, 0)))

    w_mat = jnp.transpose(weight_oihw, (2, 3, 1, 0)).astype(jnp.float32)
    w_mat = w_mat.reshape(KH * KW * Cin, Cout)
    w_big = jnp.kron(jnp.eye(Q, dtype=jnp.float32), w_mat)

    cparams = pltpu.CompilerParams(dimension_semantics=("parallel",),
                                   vmem_limit_bytes=_VMEM_LIMIT_BYTES)

    body = functools.partial(_conv_stats_kernel, nq=Q, qr=qr, w_img=W,
                             pad_rows=pad_rows)
    conv_p, stats = pl.pallas_call(
        body,
        out_shape=(jax.ShapeDtypeStruct((N, qr, P), jnp.float32),
                   jax.ShapeDtypeStruct((N, 2, P), jnp.float32)),
        grid=(N,),
        in_specs=[pl.BlockSpec((1, rows + 2 * pad_rows, Cin),
                               lambda i: (i, 0, 0)),
                  pl.BlockSpec((Q * KH * KW * Cin, P), lambda i: (0, 0))],
        out_specs=(pl.BlockSpec((1, qr, P), lambda i: (i, 0, 0)),
                   pl.BlockSpec((1, 2, P), lambda i: (i, 0, 0))),
        compiler_params=cparams,
    )(x_p, w_big)

    count = jnp.float32(N * rows)
    lane_sum = jnp.sum(stats[:, 0, :], axis=0).reshape(Q, Cout)
    lane_sq = jnp.sum(stats[:, 1, :], axis=0).reshape(Q, Cout)
    mean = jnp.sum(lane_sum, axis=0) / count
    var = jnp.maximum(jnp.sum(lane_sq, axis=0) / count - mean * mean, 0.0)
    scale = gamma.astype(jnp.float32) * lax.rsqrt(var + 1e-5)
    shift = beta.astype(jnp.float32) - mean * scale
    scale_l = jnp.tile(scale, Q).reshape(1, 1, P)
    shift_l = jnp.tile(shift, Q).reshape(1, 1, P)

    out_p = pl.pallas_call(
        _bn_relu_kernel,
        out_shape=jax.ShapeDtypeStruct((N, qr, P), jnp.float32),
        grid=(N,),
        in_specs=[pl.BlockSpec((1, qr, P), lambda i: (i, 0, 0)),
                  pl.BlockSpec((1, 1, P), lambda i: (0, 0, 0)),
                  pl.BlockSpec((1, 1, P), lambda i: (0, 0, 0))],
        out_specs=pl.BlockSpec((1, qr, P), lambda i: (i, 0, 0)),
        compiler_params=cparams,
    )(conv_p, scale_l, shift_l)

    out = out_p.reshape(N, qr, Q, Cout)
    return jnp.transpose(out, (0, 3, 2, 1)).reshape(N, Cout, H, W)


def kernel(x_nchw, weight_oihw, gamma, beta):
    return _conv_block(x_nchw, weight_oihw, gamma, beta)
